# Initial kernel scaffold; baseline (speedup 1.0000x reference)
import jax
import jax.numpy as jnp
from jax.experimental import pallas as pl


def kernel(inputs, embedding_weight):
    raise NotImplementedError("write your pallas kernel here")


# speedup vs baseline: 7.6263x; 7.6263x over previous
import jax
import jax.numpy as jnp
from jax.experimental import pallas as pl
from jax.experimental.pallas import tpu as pltpu

_NUM_EMBEDDINGS = 8192
_EMBEDDING_DIM = 64
_COMMITMENT_COST = 0.25
_TILE = 256


def _vq_body(x_ref, w_ref,
             d_ref, e_ref, q_ref, idx_ref, loss_ref, perp_ref,
             loss_acc, counts_acc):
    i = pl.program_id(0)
    nsteps = pl.num_programs(0)

    x = x_ref[...]
    w = w_ref[...]
    x2 = jnp.sum(x * x, axis=1, keepdims=True)
    e2 = jnp.sum(w * w, axis=1)[None, :]
    xw = jax.lax.dot_general(x, w, (((1,), (1,)), ((), ())),
                             preferred_element_type=jnp.float32)
    dist = x2 + e2 - 2.0 * xw
    d_ref[...] = dist

    idx = jnp.argmin(dist, axis=1)
    idx_ref[...] = idx[:, None].astype(jnp.int32)

    col = jax.lax.broadcasted_iota(jnp.int32, dist.shape, 1)
    enc = (col == idx[:, None]).astype(jnp.float32)
    e_ref[...] = enc

    q = jax.lax.dot_general(enc, w, (((1,), (0,)), ((), ())),
                            preferred_element_type=jnp.float32)
    q_ref[...] = x + (q - x)

    diff = q - x
    part = jnp.sum(diff * diff).reshape(1, 1)
    cpart = jnp.sum(enc, axis=0, keepdims=True)

    @pl.when(i == 0)
    def _init():
        loss_acc[...] = part
        counts_acc[...] = cpart

    @pl.when(i > 0)
    def _accum():
        loss_acc[...] += part
        counts_acc[...] += cpart

    @pl.when(i == nsteps - 1)
    def _finalize():
        n_tokens = nsteps * _TILE
        n_elems = jnp.float32(n_tokens * _EMBEDDING_DIM)
        loss_ref[...] = loss_acc[...] * (_COMMITMENT_COST / n_elems)
        p = counts_acc[...] / jnp.float32(n_tokens)
        perp_ref[...] = jnp.exp(-jnp.sum(p * jnp.log(p + 1e-10))).reshape(1, 1)


def kernel(inputs, embedding_weight):
    input_shape = inputs.shape
    flat = inputs.reshape(-1, _EMBEDDING_DIM)
    n = flat.shape[0]
    k = embedding_weight.shape[0]
    nsteps = n // _TILE

    out_shapes = (
        jax.ShapeDtypeStruct((n, k), jnp.float32),
        jax.ShapeDtypeStruct((n, k), jnp.float32),
        jax.ShapeDtypeStruct((n, _EMBEDDING_DIM), jnp.float32),
        jax.ShapeDtypeStruct((n, 1), jnp.int32),
        jax.ShapeDtypeStruct((1, 1), jnp.float32),
        jax.ShapeDtypeStruct((1, 1), jnp.float32),
    )
    d, e, q, idx, loss, perp = pl.pallas_call(
        _vq_body,
        grid=(nsteps,),
        in_specs=[
            pl.BlockSpec((_TILE, _EMBEDDING_DIM), lambda i: (i, 0)),
            pl.BlockSpec((k, _EMBEDDING_DIM), lambda i: (0, 0)),
        ],
        out_specs=[
            pl.BlockSpec((_TILE, k), lambda i: (i, 0)),
            pl.BlockSpec((_TILE, k), lambda i: (i, 0)),
            pl.BlockSpec((_TILE, _EMBEDDING_DIM), lambda i: (i, 0)),
            pl.BlockSpec((_TILE, 1), lambda i: (i, 0)),
            pl.BlockSpec((1, 1), lambda i: (0, 0)),
            pl.BlockSpec((1, 1), lambda i: (0, 0)),
        ],
        out_shape=out_shapes,
        scratch_shapes=[
            pltpu.VMEM((1, 1), jnp.float32),
            pltpu.VMEM((1, k), jnp.float32),
        ],
    )(flat, embedding_weight)

    vq_loss = loss[0, 0]
    quantized_st = q.reshape(input_shape)
    perplexity = perp[0, 0]
    return (vq_loss, quantized_st, perplexity, e, d, idx)
